# baseline (speedup 1.0000x reference)
import functools

import jax
import jax.numpy as jnp
from jax import lax
from jax.experimental import pallas as pl
from jax.experimental.pallas import tpu as pltpu
from jax.experimental.pallas import tpu_sc as plsc

_NUM_CORES = 2
_NUM_SUBCORES = 16
_ROWS_PER_WORKER = 8


@functools.lru_cache(maxsize=None)
def _make_gather(B, V, D):
    n_active = B // _ROWS_PER_WORKER
    mesh = plsc.VectorSubcoreMesh(core_axis_name="c", subcore_axis_name="s")

    @functools.partial(
        pl.kernel,
        out_type=jax.ShapeDtypeStruct((B, D), jnp.float32),
        mesh=mesh,
        scratch_types=[
            pltpu.VMEM((_ROWS_PER_WORKER,), jnp.int32),
            pltpu.VMEM((_ROWS_PER_WORKER, D), jnp.float32),
            pltpu.SemaphoreType.DMA,
        ],
        compiler_params=pltpu.CompilerParams(use_tc_tiling_on_sc=False),
    )
    def gather(table_hbm, idx_hbm, out_hbm, idx_v, rows_v, sem):
        wid = lax.axis_index("s") * _NUM_CORES + lax.axis_index("c")

        @pl.when(wid < n_active)
        def _():
            base = wid * _ROWS_PER_WORKER
            pltpu.sync_copy(idx_hbm.at[pl.ds(base, _ROWS_PER_WORKER)], idx_v)
            pltpu.async_copy(table_hbm.at[idx_v], rows_v, sem).wait()
            pltpu.sync_copy(rows_v, out_hbm.at[pl.ds(base, _ROWS_PER_WORKER)])

    return gather


def kernel(layer_values, ordinals):
    V, D = layer_values.shape
    (B,) = ordinals.shape
    return _make_gather(B, V, D)(layer_values, ordinals.astype(jnp.int32))


# speedup vs baseline: 1.7830x; 1.7830x over previous
import functools

import jax
import jax.numpy as jnp
from jax import lax
from jax.experimental import pallas as pl
from jax.experimental.pallas import tpu as pltpu
from jax.experimental.pallas import tpu_sc as plsc

_NUM_CORES = 2
_LANES = 16
_SUBLANES = 8
_ROWS_PER_WORKER = 8


@functools.lru_cache(maxsize=None)
def _make_gather(B, V, D):
    n_active = B // _ROWS_PER_WORKER
    mesh = plsc.VectorSubcoreMesh(core_axis_name="c", subcore_axis_name="s")

    @functools.partial(
        pl.kernel,
        out_type=jax.ShapeDtypeStruct((B, D), jnp.float32),
        mesh=mesh,
        scratch_types=[
            pltpu.VMEM((_LANES,), jnp.int32),
            pltpu.VMEM((_ROWS_PER_WORKER, _SUBLANES, D), jnp.float32),
            pltpu.VMEM((_ROWS_PER_WORKER, D), jnp.float32),
            pltpu.SemaphoreType.DMA,
        ],
        compiler_params=pltpu.CompilerParams(needs_layout_passes=False),
    )
    def gather(table_hbm, idx_hbm, out_hbm, idx_v, tiles_v, out_v, sem):
        wid = lax.axis_index("s") * _NUM_CORES + lax.axis_index("c")

        @pl.when(wid < n_active)
        def _():
            base = wid * _ROWS_PER_WORKER
            pltpu.sync_copy(idx_hbm.at[pl.ds(base, _ROWS_PER_WORKER)],
                            idx_v.at[pl.ds(0, _ROWS_PER_WORKER)])
            v = idx_v[...]
            tile_ids = lax.shift_right_logical(v, 3)
            copies = []
            for k in range(_ROWS_PER_WORKER):
                copies.append(
                    pltpu.async_copy(table_hbm.at[tile_ids[k]],
                                     tiles_v.at[k], sem))
            for c in copies:
                c.wait()
            sub = lax.bitwise_and(v, 7)
            row_ids = lax.bitwise_and(lax.iota(jnp.int32, _LANES),
                                      _ROWS_PER_WORKER - 1)
            valid = lax.iota(jnp.int32, _LANES) < _ROWS_PER_WORKER
            for c in range(D):
                col_ids = jnp.full((_LANES,), c, jnp.int32)
                col = plsc.load_gather(tiles_v, [row_ids, sub, col_ids],
                                       mask=valid)
                plsc.store_scatter(out_v, [row_ids, col_ids], col, mask=valid)
            pltpu.sync_copy(out_v, out_hbm.at[pl.ds(base, _ROWS_PER_WORKER)])

    return gather


def kernel(layer_values, ordinals):
    V, D = layer_values.shape
    (B,) = ordinals.shape
    table = jnp.reshape(layer_values, (V // _SUBLANES, _SUBLANES, D))
    return _make_gather(B, V, D)(table, ordinals.astype(jnp.int32))
